# baseline (speedup 1.0000x reference)
import functools

import jax
import jax.numpy as jnp
from jax import lax
from jax.experimental import pallas as pl
from jax.experimental.pallas import tpu as pltpu
from jax.experimental.pallas import tpu_sc as plsc

SEQ = 4
NAA = 20
VOCAB = NAA**SEQ
B = 16384
NC, NS, L = 2, 16, 16
NW = NC * NS
BPW = B // NW
GCHUNK = 128
NGATHER = BPW // GCHUNK
TCB = 4096

_mesh = plsc.VectorSubcoreMesh(
    core_axis_name="c", subcore_axis_name="s", num_cores=NC, num_subcores=NS
)


F = SEQ * NAA


def _tc_index_body(x_ref, mf_ref, o_ref):
    s = jax.lax.dot_general(
        x_ref[...],
        mf_ref[...],
        (((1,), (0,)), ((), ())),
        preferred_element_type=jnp.float32,
        precision=jax.lax.Precision.HIGHEST,
    )
    idx = jnp.clip(s, 0.0, float(VOCAB - 1)).astype(jnp.int32)
    o_ref[...] = idx.reshape(TCB)


_tc_index = pl.pallas_call(
    _tc_index_body,
    grid=(B // TCB,),
    in_specs=[
        pl.BlockSpec((TCB, F), lambda i: (i, 0)),
        pl.BlockSpec((F, 1), lambda i: (0, 0)),
    ],
    out_specs=pl.BlockSpec((TCB,), lambda i: (i,)),
    out_shape=jax.ShapeDtypeStruct((B,), jnp.int32),
)


@functools.partial(
    pl.kernel,
    out_type=jax.ShapeDtypeStruct((B,), jnp.float32),
    mesh=_mesh,
    compiler_params=pltpu.CompilerParams(needs_layout_passes=False),
    scratch_types=[
        pltpu.VMEM_SHARED((VOCAB,), jnp.float32),
        pltpu.VMEM((BPW,), jnp.int32),
        pltpu.VMEM((BPW,), jnp.float32),
        pltpu.SemaphoreType.DMA,
        pltpu.SemaphoreType.DMA,
    ],
)
def _sc_gather(idx_hbm, fit_hbm, out_hbm, fit_s, idx_v, val_v, si, sg):
    cid = lax.axis_index("c")
    sid = lax.axis_index("s")
    base = (sid * NC + cid) * BPW

    idx_cp = pltpu.async_copy(idx_hbm.at[pl.ds(base, BPW)], idx_v, si)

    @pl.when(sid == 0)
    def _stage_table():
        pltpu.sync_copy(fit_hbm, fit_s)

    plsc.subcore_barrier()
    idx_cp.wait()

    gathers = [
        pltpu.async_copy(
            fit_s.at[idx_v.at[pl.ds(t * GCHUNK, GCHUNK)]],
            val_v.at[pl.ds(t * GCHUNK, GCHUNK)],
            sg,
        )
        for t in range(NGATHER)
    ]
    for g in gathers:
        g.wait()
    pltpu.sync_copy(val_v, out_hbm.at[pl.ds(base, BPW)])


def kernel(x, fitnesses, mult_factor):
    idx = _tc_index(x.reshape(B, F), mult_factor.reshape(F, 1))
    return _sc_gather(idx, fitnesses.reshape(VOCAB))
